# Initial kernel scaffold; baseline (speedup 1.0000x reference)
import jax
import jax.numpy as jnp
from jax.experimental import pallas as pl


def kernel(outputs1, outputs2, outputs3, outputs4, outputs5, outputs6, outputs7, mimic, targets, n_test):
    raise NotImplementedError("write your pallas kernel here")


# speedup vs baseline: 122.8115x; 122.8115x over previous
import functools

import jax
import jax.numpy as jnp
from jax import lax
from jax.experimental import pallas as pl
from jax.experimental.pallas import tpu as pltpu
from jax.experimental.pallas import tpu_sc as plsc

B = 128
N = 32768
T = 2.0

_NC = 2
_NS = 16
_NW = _NC * _NS
_RPW = B // _NW
_L = 16
_UNROLL = 8
_VECS = N // _L
_STEPS = _VECS // _UNROLL

_NEG = -3e38


def _row_top2(rowbuf, parity):
    init = tuple(jnp.full((_L,), _NEG, jnp.float32) for _ in range(2 * _UNROLL))

    def body(j, accs):
        base = j * (_UNROLL * _L)
        out = list(accs)
        for u in range(_UNROLL):
            v = rowbuf[parity, pl.ds(base + u * _L, _L)]
            m1 = accs[2 * u]
            m2 = accs[2 * u + 1]
            out[2 * u] = jnp.maximum(m1, v)
            out[2 * u + 1] = jnp.maximum(m2, jnp.minimum(m1, v))
        return tuple(out)

    accs = lax.fori_loop(0, _STEPS, body, init, unroll=1)

    m1, m2 = accs[0], accs[1]
    for u in range(1, _UNROLL):
        a1, a2 = accs[2 * u], accs[2 * u + 1]
        m2 = jnp.maximum(jnp.maximum(m2, a2), jnp.minimum(m1, a1))
        m1 = jnp.maximum(m1, a1)

    g1 = jnp.max(m1)
    eq = m1 == g1
    ncnt = jnp.max(plsc.all_reduce_population_count(eq))
    rest = jnp.max(jnp.where(eq, jnp.float32(_NEG), m1))
    g2_unique = jnp.maximum(rest, jnp.max(m2))
    g2 = jnp.where(ncnt >= 2, g1, g2_unique)
    return g1, g2, m1


def _sc_body(o1, o2, o3, o4, o5, o6, o7, mim, tgt_hbm,
             out_thr, out_max, rowbuf, tgtbuf, stage, sem):
    arrs = [o1, o2, o3, o4, o5, o6, o7, mim]
    wid = lax.axis_index("s") * _NC + lax.axis_index("c")
    row0 = wid * _RPW

    pltpu.sync_copy(tgt_hbm, tgtbuf)

    lanes = lax.iota(jnp.int32, _L)
    lmax = jnp.full((_L,), _NEG, jnp.float32)

    def start(t):
        r, a = divmod(t, 8)
        return pltpu.async_copy(arrs[a].at[row0 + r], rowbuf.at[t % 2], sem)

    copies = {0: start(0)}
    for r in range(_RPW):
        tval = plsc.load_gather(tgtbuf, [jnp.full((_L,), row0 + r, jnp.int32)])
        d = jnp.full((_L,), _NEG, jnp.float32)
        for a in range(8):
            t = r * 8 + a
            if t + 1 < _RPW * 8:
                copies[t + 1] = start(t + 1)
            copies.pop(t).wait()
            g1, g2, m1 = _row_top2(rowbuf, t % 2)
            tv = plsc.load_gather(
                rowbuf, [jnp.full((_L,), t % 2, jnp.int32), tval])
            margin = jnp.where(tv == g1, g1 - g2, jnp.float32(0.0))
            d = jnp.where(lanes == a, margin, d)
            if a < 7:
                lmax = jnp.maximum(lmax, m1)
        mx = jnp.max(d)
        e = jnp.exp((d - mx) * jnp.float32(1.0 / T))
        e = e / jnp.broadcast_to(jnp.sum(e), (_L,))
        stage[0] = e
        pltpu.sync_copy(stage.at[0], out_thr.at[row0 + r])

    stage[0] = lmax
    pltpu.sync_copy(stage.at[0], out_max.at[wid])


@jax.jit
def _run(o1, o2, o3, o4, o5, o6, o7, mim, tgt):
    mesh = plsc.VectorSubcoreMesh(core_axis_name="c", subcore_axis_name="s")
    fn = functools.partial(
        pl.kernel,
        mesh=mesh,
        compiler_params=pltpu.CompilerParams(needs_layout_passes=False),
        out_type=[
            jax.ShapeDtypeStruct((B, _L), jnp.float32),
            jax.ShapeDtypeStruct((_NW, _L), jnp.float32),
        ],
        scratch_types=[
            pltpu.VMEM((2, N), jnp.float32),
            pltpu.VMEM((B,), jnp.int32),
            pltpu.VMEM((1, _L), jnp.float32),
            pltpu.SemaphoreType.DMA,
        ],
    )(_sc_body)
    return fn(o1, o2, o3, o4, o5, o6, o7, mim, tgt)


def kernel(outputs1, outputs2, outputs3, outputs4, outputs5, outputs6,
           outputs7, mimic, targets, n_test):
    del n_test
    thr, pmax = _run(outputs1, outputs2, outputs3, outputs4, outputs5,
                     outputs6, outputs7, mimic, targets.astype(jnp.int32))
    return jnp.max(pmax), thr[:, :8]


# speedup vs baseline: 188.0878x; 1.5315x over previous
import functools

import jax
import jax.numpy as jnp
from jax import lax
from jax.experimental import pallas as pl
from jax.experimental.pallas import tpu as pltpu
from jax.experimental.pallas import tpu_sc as plsc

B = 128
N = 32768
T = 2.0

_NC = 2
_NS = 16
_NW = _NC * _NS
_RPW = B // _NW
_L = 16
_UNROLL = 8
_STEPS = N // (_L * _UNROLL)

_NEG = -3e38

_SC_ARRS = (5, 6, 7)
_N_TC = 5
_TILE = 2048
_GRID = N // _TILE


def _row_top2(rowbuf, parity):
    init = tuple(jnp.full((_L,), _NEG, jnp.float32) for _ in range(2 * _UNROLL))

    def body(j, accs):
        base = j * (_UNROLL * _L)
        out = list(accs)
        for u in range(_UNROLL):
            v = rowbuf[parity, pl.ds(base + u * _L, _L)]
            m1 = accs[2 * u]
            m2 = accs[2 * u + 1]
            out[2 * u] = jnp.maximum(m1, v)
            out[2 * u + 1] = jnp.maximum(m2, jnp.minimum(m1, v))
        return tuple(out)

    accs = lax.fori_loop(0, _STEPS, body, init, unroll=1)

    m1, m2 = accs[0], accs[1]
    for u in range(1, _UNROLL):
        a1, a2 = accs[2 * u], accs[2 * u + 1]
        m2 = jnp.maximum(jnp.maximum(m2, a2), jnp.minimum(m1, a1))
        m1 = jnp.maximum(m1, a1)

    g1 = jnp.max(m1)
    eq = m1 == g1
    ncnt = jnp.max(plsc.all_reduce_population_count(eq))
    rest = jnp.max(jnp.where(eq, jnp.float32(_NEG), m1))
    g2_unique = jnp.maximum(rest, jnp.max(m2))
    g2 = jnp.where(ncnt >= 2, g1, g2_unique)
    return g1, g2, m1


def _sc_body(a5, a6, a7, tgt_hbm, out_marg, out_max,
             rowbuf, tgtbuf, stage, sem):
    arrs = [a5, a6, a7]
    wid = lax.axis_index("s") * _NC + lax.axis_index("c")
    row0 = wid * _RPW
    ntask = _RPW * len(arrs)

    pltpu.sync_copy(tgt_hbm, tgtbuf)

    lanes = lax.iota(jnp.int32, _L)
    lmax = jnp.full((_L,), _NEG, jnp.float32)

    def start(t):
        r, i = divmod(t, len(arrs))
        return pltpu.async_copy(arrs[i].at[row0 + r], rowbuf.at[t % 2], sem)

    copies = {0: start(0)}
    for r in range(_RPW):
        tval = plsc.load_gather(tgtbuf, [jnp.full((_L,), row0 + r, jnp.int32)])
        d = jnp.full((_L,), 0.0, jnp.float32)
        for i, aid in enumerate(_SC_ARRS):
            t = r * len(arrs) + i
            if t + 1 < ntask:
                copies[t + 1] = start(t + 1)
            copies.pop(t).wait()
            g1, g2, m1 = _row_top2(rowbuf, t % 2)
            tv = plsc.load_gather(
                rowbuf, [jnp.full((_L,), t % 2, jnp.int32), tval])
            margin = jnp.where(tv == g1, g1 - g2, jnp.float32(0.0))
            d = jnp.where(lanes == aid, margin, d)
            if aid < 7:
                lmax = jnp.maximum(lmax, m1)
        stage[0] = d
        pltpu.sync_copy(stage.at[0], out_marg.at[row0 + r])

    stage[0] = lmax
    pltpu.sync_copy(stage.at[0], out_max.at[wid])


def _tc_body(t1, t2, t3, t4, t5, tgt, marg, rowmax, m1s, m2s, tvs):
    j = pl.program_id(0)
    tiles = [t1, t2, t3, t4, t5]

    @pl.when(j == 0)
    def _init():
        m1s[...] = jnp.full((B, 8), _NEG, jnp.float32)
        m2s[...] = jnp.full((B, 8), _NEG, jnp.float32)
        tvs[...] = jnp.zeros((B, 8), jnp.float32)

    col = lax.broadcasted_iota(jnp.int32, (B, _TILE), 1)
    gcol = col + j * _TILE
    tv_tgt = tgt[...]
    for a in range(_N_TC):
        x = tiles[a][...]
        m1_t = jnp.max(x, axis=1, keepdims=True)
        eq = x == m1_t
        fi = jnp.min(jnp.where(eq, col, jnp.int32(1 << 30)), axis=1,
                     keepdims=True)
        m2_t = jnp.max(jnp.where(col == fi, jnp.float32(_NEG), x), axis=1,
                       keepdims=True)
        tv_t = jnp.sum(jnp.where(gcol == tv_tgt, x, jnp.float32(0.0)),
                       axis=1, keepdims=True)
        o1 = m1s[:, a:a + 1]
        o2 = m2s[:, a:a + 1]
        m2s[:, a:a + 1] = jnp.maximum(jnp.maximum(o2, m2_t),
                                      jnp.minimum(o1, m1_t))
        m1s[:, a:a + 1] = jnp.maximum(o1, m1_t)
        tvs[:, a:a + 1] = tvs[:, a:a + 1] + tv_t

    @pl.when(j == _GRID - 1)
    def _fin():
        m1 = m1s[...]
        marg[...] = jnp.where(tvs[...] == m1, m1 - m2s[...],
                              jnp.float32(0.0))
        rowmax[...] = m1


def _combine_body(tc_m, tc_rm, sc_m, sc_lm, thr, gmax):
    cols = lax.broadcasted_iota(jnp.int32, (B, 8), 1)
    cm = jnp.where(cols < _N_TC, tc_m[...], sc_m[...])
    mx = jnp.max(cm, axis=1, keepdims=True)
    e = jnp.exp((cm - mx) * jnp.float32(1.0 / T))
    thr[...] = e / jnp.sum(e, axis=1, keepdims=True)
    g = jnp.maximum(jnp.max(tc_rm[...]), jnp.max(sc_lm[...]))
    gmax[...] = jnp.full((8, 128), g, jnp.float32)


@jax.jit
def _run(o1, o2, o3, o4, o5, o6, o7, mim, tgt):
    mesh = plsc.VectorSubcoreMesh(core_axis_name="c", subcore_axis_name="s")
    sc_fn = functools.partial(
        pl.kernel,
        mesh=mesh,
        compiler_params=pltpu.CompilerParams(needs_layout_passes=False),
        out_type=[
            jax.ShapeDtypeStruct((B, _L), jnp.float32),
            jax.ShapeDtypeStruct((_NW, _L), jnp.float32),
        ],
        scratch_types=[
            pltpu.VMEM((2, N), jnp.float32),
            pltpu.VMEM((B,), jnp.int32),
            pltpu.VMEM((1, _L), jnp.float32),
            pltpu.SemaphoreType.DMA,
        ],
    )(_sc_body)
    sc_marg, sc_lmax = sc_fn(o6, o7, mim, tgt)

    blk = pl.BlockSpec((B, _TILE), lambda j: (0, j))
    tc_marg, tc_rowmax = pl.pallas_call(
        _tc_body,
        grid=(_GRID,),
        in_specs=[blk] * _N_TC + [pl.BlockSpec((B, 1), lambda j: (0, 0))],
        out_specs=[pl.BlockSpec((B, 8), lambda j: (0, 0))] * 2,
        out_shape=[jax.ShapeDtypeStruct((B, 8), jnp.float32)] * 2,
        scratch_shapes=[pltpu.VMEM((B, 8), jnp.float32)] * 3,
    )(o1, o2, o3, o4, o5, tgt.reshape(B, 1))

    thr, gmax = pl.pallas_call(
        _combine_body,
        in_specs=[
            pl.BlockSpec((B, 8), lambda: (0, 0)),
            pl.BlockSpec((B, 8), lambda: (0, 0)),
            pl.BlockSpec((B, 8), lambda: (0, 0)),
            pl.BlockSpec((_NW, _L), lambda: (0, 0)),
        ],
        out_specs=[
            pl.BlockSpec((B, 8), lambda: (0, 0)),
            pl.BlockSpec((8, 128), lambda: (0, 0)),
        ],
        out_shape=[
            jax.ShapeDtypeStruct((B, 8), jnp.float32),
            jax.ShapeDtypeStruct((8, 128), jnp.float32),
        ],
    )(tc_marg, tc_rowmax, sc_marg[:, :8], sc_lmax)
    return thr, gmax


def kernel(outputs1, outputs2, outputs3, outputs4, outputs5, outputs6,
           outputs7, mimic, targets, n_test):
    del n_test
    thr, gmax = _run(outputs1, outputs2, outputs3, outputs4, outputs5,
                     outputs6, outputs7, mimic, targets.astype(jnp.int32))
    return gmax[0, 0], thr
